# baseline (device time: 19188 ns/iter reference)
import jax
import jax.numpy as jnp
from jax import lax
from jax.experimental import pallas as pl
from jax.experimental.pallas import tpu as pltpu

N_DEV = 8
B, Sq, D, Hq, Hkv, Dh = 2, 128, 512, 8, 2, 64
G = Hq // Hkv
SCALE = 0.125
QSCALE = 16.0


def kernel(x, Wq, Wo, K_ext, V_ext):
    c = K_ext.shape[1]
    skv = N_DEV * c

    def body(x_hbm, wq_hbm, wo_hbm, k_hbm, v_hbm, out_hbm,
             x_ref, wq_ref, wo_ref, k_ref, v_ref, out_vmem,
             stagek_ref, stagev_ref, kgat_ref, vgat_ref, p_ref, copy_sems, send_sems, recv_sems):
        me = lax.axis_index("i")

        cp_k = pltpu.make_async_copy(k_hbm, k_ref, copy_sems.at[0])
        cp_v = pltpu.make_async_copy(v_hbm, v_ref, copy_sems.at[1])
        cp_x = pltpu.make_async_copy(x_hbm, x_ref, copy_sems.at[2])
        cp_wq = pltpu.make_async_copy(wq_hbm, wq_ref, copy_sems.at[3])
        cp_wo = pltpu.make_async_copy(wo_hbm, wo_ref, copy_sems.at[4])
        for cp in (cp_k, cp_v, cp_x, cp_wq, cp_wo):
            cp.start()

        barrier_sem = pltpu.get_barrier_semaphore()
        for off in range(1, N_DEV):
            pl.semaphore_signal(
                barrier_sem, inc=1,
                device_id=((me + off) % N_DEV,),
                device_id_type=pl.DeviceIdType.MESH,
            )

        cp_k.wait()
        cp_v.wait()
        stagek_ref[...] = jnp.rint(
            jnp.clip(k_ref[...].reshape(B, c, Hkv * Dh) * QSCALE, -127, 127)
        ).astype(jnp.int8)

        pl.semaphore_wait(barrier_sem, N_DEV - 1)

        def make_rdma_k(off):
            return pltpu.make_async_remote_copy(
                src_ref=stagek_ref,
                dst_ref=kgat_ref.at[:, (off - 1) * c:off * c, :],
                send_sem=send_sems.at[off - 1],
                recv_sem=recv_sems.at[off - 1],
                device_id=((me + off) % N_DEV,),
                device_id_type=pl.DeviceIdType.MESH,
            )

        def make_rdma_v(off):
            return pltpu.make_async_remote_copy(
                src_ref=stagev_ref,
                dst_ref=vgat_ref.at[:, (off - 1) * c:off * c, :],
                send_sem=send_sems.at[N_DEV - 1 + off - 1],
                recv_sem=recv_sems.at[N_DEV - 1 + off - 1],
                device_id=((me + off) % N_DEV,),
                device_id_type=pl.DeviceIdType.MESH,
            )

        SEND_ORDER = (6, 2, 5, 7, 1, 3, 4)
        for off in SEND_ORDER:
            make_rdma_k(off).start()
        stagev_ref[...] = v_ref[...].reshape(B, c, Hkv * Dh).astype(jnp.bfloat16)
        for off in SEND_ORDER:
            make_rdma_v(off).start()
        kgat_ref[:, (N_DEV - 1) * c:, :] = stagek_ref[...]
        vgat_ref[:, (N_DEV - 1) * c:, :] = stagev_ref[...]

        cp_x.wait()
        cp_wq.wait()
        q2d = lax.dot_general(
            x_ref[...].reshape(B * Sq, D).astype(jnp.bfloat16),
            wq_ref[...].astype(jnp.bfloat16),
            (((1,), (0,)), ((), ())),
            preferred_element_type=jnp.float32,
        )
        qstack = {}
        for b in range(B):
            qb = q2d[b * Sq:(b + 1) * Sq, :]
            for k in range(Hkv):
                qstack[(b, k)] = jnp.concatenate(
                    [qb[:, (G * k + g) * Dh:(G * k + g + 1) * Dh]
                     for g in range(G)], axis=0,
                ).astype(jnp.bfloat16)

        lsum = {key: jnp.zeros((G * Sq, 1), jnp.float32) for key in qstack}

        def score(slot):
            for b in range(B):
                for k in range(Hkv):
                    kc = kgat_ref[b, slot * c:(slot + 1) * c,
                                  k * Dh:(k + 1) * Dh].astype(jnp.bfloat16)
                    p = jnp.exp(lax.dot_general(
                        qstack[(b, k)], kc,
                        (((1,), (1,)), ((), ())),
                        preferred_element_type=jnp.float32,
                    ) * (SCALE / QSCALE))
                    lsum[(b, k)] = lsum[(b, k)] + jnp.sum(p, -1, keepdims=True)
                    p_ref[b * Hkv + k, :, slot * c:(slot + 1) * c] = (
                        p.astype(jnp.bfloat16))

        score(N_DEV - 1)
        for s in range(N_DEV - 1):
            make_rdma_k(s + 1).wait_recv()
            score(s)
        for s in range(N_DEV - 1):
            make_rdma_v(s + 1).wait_recv()

        cp_wo.wait()
        wo = wo_ref[...].astype(jnp.bfloat16)
        for b in range(B):
            cols = []
            for k in range(Hkv):
                vall = vgat_ref[b, :, k * Dh:(k + 1) * Dh]
                o = lax.dot_general(
                    p_ref[b * Hkv + k], vall,
                    (((1,), (0,)), ((), ())),
                    preferred_element_type=jnp.float32,
                ) / lsum[(b, k)]
                for g in range(G):
                    cols.append(o[g * Sq:(g + 1) * Sq, :])
            attn = jnp.concatenate(cols, axis=1).astype(jnp.bfloat16)
            out_vmem[b] = lax.dot_general(
                attn, wo, (((1,), (0,)), ((), ())),
                preferred_element_type=jnp.float32,
            ).astype(jnp.bfloat16)

        cp_out = pltpu.make_async_copy(out_vmem, out_hbm, copy_sems.at[5])
        cp_out.start()
        cp_out.wait()

        for off in range(1, N_DEV):
            make_rdma_k(off).wait_send()
            make_rdma_v(off).wait_send()

    return pl.pallas_call(
        body,
        out_shape=jax.ShapeDtypeStruct((B, Sq, D), jnp.bfloat16),
        in_specs=[pl.BlockSpec(memory_space=pl.ANY)] * 5,
        out_specs=pl.BlockSpec(memory_space=pl.ANY),
        scratch_shapes=[
            pltpu.VMEM((B, Sq, D), jnp.float32),
            pltpu.VMEM((D, D), jnp.float32),
            pltpu.VMEM((D, D), jnp.float32),
            pltpu.VMEM((B, c, Hkv, Dh), jnp.float32),
            pltpu.VMEM((B, c, Hkv, Dh), jnp.float32),
            pltpu.VMEM((B, Sq, D), jnp.bfloat16),
            pltpu.VMEM((B, c, Hkv * Dh), jnp.int8),
            pltpu.VMEM((B, c, Hkv * Dh), jnp.bfloat16),
            pltpu.VMEM((B, N_DEV * c, Hkv * Dh), jnp.int8),
            pltpu.VMEM((B, N_DEV * c, Hkv * Dh), jnp.bfloat16),
            pltpu.VMEM((B * Hkv, G * Sq, N_DEV * c), jnp.bfloat16),
            pltpu.SemaphoreType.DMA((6,)),
            pltpu.SemaphoreType.DMA((2 * (N_DEV - 1),)),
            pltpu.SemaphoreType.DMA((2 * (N_DEV - 1),)),
        ],
        compiler_params=pltpu.CompilerParams(collective_id=0),
    )(x, Wq, Wo, K_ext, V_ext)


# device time: 17928 ns/iter; 1.0703x vs baseline; 1.0703x over previous
import jax
import jax.numpy as jnp
from jax import lax
from jax.experimental import pallas as pl
from jax.experimental.pallas import tpu as pltpu

N_DEV = 8
B, Sq, D, Hq, Hkv, Dh = 2, 128, 512, 8, 2, 64
G = Hq // Hkv
SCALE = 0.125
QSCALE = 16.0


def kernel(x, Wq, Wo, K_ext, V_ext):
    c = K_ext.shape[1]
    skv = N_DEV * c

    def body(x_hbm, wq_hbm, wo_hbm, k_hbm, v_hbm, out_hbm,
             x_ref, wq_ref, wo_ref, k_ref, v_ref, out_vmem,
             stagek_ref, stagev_ref, kgat_ref, vgat_ref, p_ref, copy_sems, send_sems, recv_sems):
        me = lax.axis_index("i")

        cp_k = pltpu.make_async_copy(k_hbm, k_ref, copy_sems.at[0])
        cp_v = pltpu.make_async_copy(v_hbm, v_ref, copy_sems.at[1])
        cp_x = pltpu.make_async_copy(x_hbm, x_ref, copy_sems.at[2])
        cp_wq = pltpu.make_async_copy(wq_hbm, wq_ref, copy_sems.at[3])
        cp_wo = pltpu.make_async_copy(wo_hbm, wo_ref, copy_sems.at[4])
        for cp in (cp_k, cp_v, cp_x, cp_wq, cp_wo):
            cp.start()

        barrier_sem = pltpu.get_barrier_semaphore()
        for off in range(1, N_DEV):
            pl.semaphore_signal(
                barrier_sem, inc=1,
                device_id=((me + off) % N_DEV,),
                device_id_type=pl.DeviceIdType.MESH,
            )

        cp_k.wait()
        cp_v.wait()
        stagek_ref[...] = jnp.rint(
            jnp.clip(k_ref[...].reshape(B, c, Hkv * Dh) * QSCALE, -127, 127)
        ).astype(jnp.int8)
        stagev_ref[...] = v_ref[...].reshape(B, c, Hkv * Dh).astype(jnp.bfloat16)
        kgat_ref[:, (N_DEV - 1) * c:, :] = stagek_ref[...]
        vgat_ref[:, (N_DEV - 1) * c:, :] = stagev_ref[...]

        pl.semaphore_wait(barrier_sem, N_DEV - 1)

        def make_rdma_k(off):
            return pltpu.make_async_remote_copy(
                src_ref=stagek_ref,
                dst_ref=kgat_ref.at[:, (off - 1) * c:off * c, :],
                send_sem=send_sems.at[off - 1],
                recv_sem=recv_sems.at[off - 1],
                device_id=((me + off) % N_DEV,),
                device_id_type=pl.DeviceIdType.MESH,
            )

        def make_rdma_v(off):
            return pltpu.make_async_remote_copy(
                src_ref=stagev_ref,
                dst_ref=vgat_ref.at[:, (off - 1) * c:off * c, :],
                send_sem=send_sems.at[N_DEV - 1 + off - 1],
                recv_sem=recv_sems.at[N_DEV - 1 + off - 1],
                device_id=((me + off) % N_DEV,),
                device_id_type=pl.DeviceIdType.MESH,
            )

        for off in range(1, N_DEV):
            make_rdma_k(off).start()
        for off in range(1, N_DEV):
            make_rdma_v(off).start()

        cp_x.wait()
        cp_wq.wait()
        q2d = lax.dot_general(
            x_ref[...].reshape(B * Sq, D).astype(jnp.bfloat16),
            wq_ref[...].astype(jnp.bfloat16),
            (((1,), (0,)), ((), ())),
            preferred_element_type=jnp.float32,
        )
        qstack = {}
        for b in range(B):
            qb = q2d[b * Sq:(b + 1) * Sq, :]
            for k in range(Hkv):
                qstack[(b, k)] = jnp.concatenate(
                    [qb[:, (G * k + g) * Dh:(G * k + g + 1) * Dh]
                     for g in range(G)], axis=0,
                ).astype(jnp.bfloat16)

        lsum = {key: jnp.zeros((G * Sq, 1), jnp.float32) for key in qstack}

        def score(slot):
            for b in range(B):
                for k in range(Hkv):
                    kc = kgat_ref[b, slot * c:(slot + 1) * c,
                                  k * Dh:(k + 1) * Dh].astype(jnp.bfloat16)
                    p = jnp.exp(lax.dot_general(
                        qstack[(b, k)], kc,
                        (((1,), (1,)), ((), ())),
                        preferred_element_type=jnp.float32,
                    ) * (SCALE / QSCALE))
                    lsum[(b, k)] = lsum[(b, k)] + jnp.sum(p, -1, keepdims=True)
                    p_ref[b * Hkv + k, :, slot * c:(slot + 1) * c] = (
                        p.astype(jnp.bfloat16))

        score(N_DEV - 1)
        for off in (1, 3, 4, 2, 5, 7, 6):
            make_rdma_k(off).wait_recv()
            score(off - 1)
        for s in range(N_DEV - 1):
            make_rdma_v(s + 1).wait_recv()

        cp_wo.wait()
        wo = wo_ref[...].astype(jnp.bfloat16)
        for b in range(B):
            cols = []
            for k in range(Hkv):
                vall = vgat_ref[b, :, k * Dh:(k + 1) * Dh]
                o = lax.dot_general(
                    p_ref[b * Hkv + k], vall,
                    (((1,), (0,)), ((), ())),
                    preferred_element_type=jnp.float32,
                ) / lsum[(b, k)]
                for g in range(G):
                    cols.append(o[g * Sq:(g + 1) * Sq, :])
            attn = jnp.concatenate(cols, axis=1).astype(jnp.bfloat16)
            out_vmem[b] = lax.dot_general(
                attn, wo, (((1,), (0,)), ((), ())),
                preferred_element_type=jnp.float32,
            ).astype(jnp.bfloat16)

        cp_out = pltpu.make_async_copy(out_vmem, out_hbm, copy_sems.at[5])
        cp_out.start()
        cp_out.wait()

        for off in range(1, N_DEV):
            make_rdma_k(off).wait_send()
            make_rdma_v(off).wait_send()

    return pl.pallas_call(
        body,
        out_shape=jax.ShapeDtypeStruct((B, Sq, D), jnp.bfloat16),
        in_specs=[pl.BlockSpec(memory_space=pl.ANY)] * 5,
        out_specs=pl.BlockSpec(memory_space=pl.ANY),
        scratch_shapes=[
            pltpu.VMEM((B, Sq, D), jnp.float32),
            pltpu.VMEM((D, D), jnp.float32),
            pltpu.VMEM((D, D), jnp.float32),
            pltpu.VMEM((B, c, Hkv, Dh), jnp.float32),
            pltpu.VMEM((B, c, Hkv, Dh), jnp.float32),
            pltpu.VMEM((B, Sq, D), jnp.bfloat16),
            pltpu.VMEM((B, c, Hkv * Dh), jnp.int8),
            pltpu.VMEM((B, c, Hkv * Dh), jnp.bfloat16),
            pltpu.VMEM((B, N_DEV * c, Hkv * Dh), jnp.int8),
            pltpu.VMEM((B, N_DEV * c, Hkv * Dh), jnp.bfloat16),
            pltpu.VMEM((B * Hkv, G * Sq, N_DEV * c), jnp.bfloat16),
            pltpu.SemaphoreType.DMA((6,)),
            pltpu.SemaphoreType.DMA((2 * (N_DEV - 1),)),
            pltpu.SemaphoreType.DMA((2 * (N_DEV - 1),)),
        ],
        compiler_params=pltpu.CompilerParams(collective_id=0),
    )(x, Wq, Wo, K_ext, V_ext)
